# baseline (speedup 1.0000x reference)
import functools

import jax
import jax.numpy as jnp
from jax import lax
from jax.experimental import pallas as pl
from jax.experimental.pallas import tpu as pltpu
from jax.experimental.pallas import tpu_sc as plsc

H = 2048
W = 2048
BINS = 256
LANES = 16
NC = 2
NS = 16
NW = NC * NS
ROWS_W = H // NW
CROWS = 16
NCHUNK = ROWS_W // CROWS


def _hist_body(img_hbm, out_hbm, buf, hist_ln, sem0, sem1):
    cid = lax.axis_index("c")
    sid = lax.axis_index("s")
    wid = cid * NS + sid
    row0 = wid * ROWS_W

    zeros16 = jnp.zeros((LANES,), jnp.int32)
    ones16 = jnp.ones((LANES,), jnp.int32)
    lane_base = lax.iota(jnp.int32, LANES) * BINS

    @plsc.parallel_loop(0, LANES * BINS, step=LANES, unroll=8)
    def _zero(j):
        hist_ln[pl.ds(j, LANES)] = zeros16

    sems = (sem0, sem1)

    def start(k):
        return pltpu.async_copy(
            img_hbm.at[pl.ds(row0 + k * CROWS, CROWS), :],
            buf.at[k % 2], sems[k % 2])

    cp = start(0)
    for k in range(NCHUNK):
        b = k % 2
        nxt = start(k + 1) if k + 1 < NCHUNK else None
        cp.wait()

        @plsc.parallel_loop(0, CROWS * W, step=LANES, unroll=16)
        def _scat(off):
            v = buf[b, off >> 11, pl.ds(off & (W - 1), LANES)]
            plsc.addupdate_scatter(hist_ln, [lane_base | v], ones16)

        cp = nxt

    pltpu.sync_copy(hist_ln, out_hbm.at[wid])


@functools.cache
def _hist_sc():
    return pl.kernel(
        _hist_body,
        out_type=jax.ShapeDtypeStruct((NW, LANES * BINS), jnp.int32),
        mesh=plsc.VectorSubcoreMesh(core_axis_name="c", subcore_axis_name="s"),
        compiler_params=pltpu.CompilerParams(needs_layout_passes=False),
        scratch_types=[
            pltpu.VMEM((2, CROWS, W), jnp.int32),
            pltpu.VMEM((LANES * BINS,), jnp.int32),
            pltpu.SemaphoreType.DMA,
            pltpu.SemaphoreType.DMA,
        ],
    )


TILE = 2048


def _finish_body(part_ref, img_ref, thresh_ref, out_ref):
    step = pl.program_id(0)

    @pl.when(step == 0)
    def _():
        acc = part_ref[:, :BINS]
        for l in range(1, LANES):
            acc = acc + part_ref[:, l * BINS:(l + 1) * BINS]
        hist = jnp.sum(acc, axis=0, keepdims=True)
        vals = lax.broadcasted_iota(jnp.int32, (1, BINS), 1)
        fc = hist * vals
        rows = lax.broadcasted_iota(jnp.int32, (BINS, BINS), 0)
        cols = lax.broadcasted_iota(jnp.int32, (BINS, BINS), 1)
        mask = cols <= rows
        zero = jnp.zeros((BINS, BINS), jnp.int32)
        num_bk = jnp.sum(jnp.where(mask, hist, zero), axis=1,
                         keepdims=True)
        fc_bk = jnp.sum(jnp.where(mask, fc, zero), axis=1, keepdims=True)
        num_wh = jnp.sum(hist) - num_bk
        fc_wh = jnp.sum(fc) - fc_bk
        mean_bk = fc_bk / num_bk
        mean_wh = fc_wh / num_wh
        var = (num_bk.astype(jnp.float32) * num_wh.astype(jnp.float32)
               * (mean_bk - mean_wh) ** 2)
        var = jnp.where(jnp.isnan(var), jnp.float32(0), var)
        vmax = jnp.max(var)
        tidx = lax.broadcasted_iota(jnp.int32, (BINS, 1), 0)
        thresh_ref[0, 0] = jnp.min(
            jnp.where(var == vmax, tidx, jnp.int32(BINS)))

    t = thresh_ref[0, 0]
    out_ref[...] = jnp.where(img_ref[...] <= t, jnp.int32(0), jnp.int32(255))


_finish = pl.pallas_call(
    _finish_body,
    grid=(H // TILE,),
    in_specs=[
        pl.BlockSpec((NW, LANES * BINS), lambda i: (0, 0)),
        pl.BlockSpec((TILE, W), lambda i: (i, 0)),
    ],
    out_specs=[
        pl.BlockSpec(memory_space=pltpu.SMEM),
        pl.BlockSpec((TILE, W), lambda i: (i, 0)),
    ],
    out_shape=[
        jax.ShapeDtypeStruct((1, 1), jnp.int32),
        jax.ShapeDtypeStruct((H, W), jnp.int32),
    ],
    compiler_params=pltpu.CompilerParams(
        dimension_semantics=("arbitrary",)),
)


def kernel(img_HxW):
    partials = _hist_sc()(img_HxW)
    thresh2d, bin_img = _finish(partials, img_HxW)
    return thresh2d[0, 0], bin_img


# speedup vs baseline: 1.0635x; 1.0635x over previous
import functools

import jax
import jax.numpy as jnp
from jax import lax
from jax.experimental import pallas as pl
from jax.experimental.pallas import tpu as pltpu
from jax.experimental.pallas import tpu_sc as plsc

H = 2048
W = 2048
BINS = 256
LANES = 16
NC = 2
NS = 16
NW = NC * NS
ROWS_W = H // NW
CROWS = 8
NCHUNK = ROWS_W // CROWS


def _hist_body(img_hbm, out_hbm, buf, hist_ln, sem0, sem1):
    cid = lax.axis_index("c")
    sid = lax.axis_index("s")
    wid = cid * NS + sid
    row0 = wid * ROWS_W

    zeros16 = jnp.zeros((LANES,), jnp.int32)
    ones16 = jnp.ones((LANES,), jnp.int32)
    lane_base = lax.iota(jnp.int32, LANES) * BINS

    @plsc.parallel_loop(0, LANES * BINS, step=LANES, unroll=8)
    def _zero(j):
        hist_ln[pl.ds(j, LANES)] = zeros16

    sems = (sem0, sem1)

    def start(k):
        return pltpu.async_copy(
            img_hbm.at[pl.ds(row0 + k * CROWS, CROWS), :],
            buf.at[k % 2], sems[k % 2])

    cp = start(0)
    for k in range(NCHUNK):
        b = k % 2
        nxt = start(k + 1) if k + 1 < NCHUNK else None
        cp.wait()

        @plsc.parallel_loop(0, CROWS * W, step=LANES, unroll=16)
        def _scat(off):
            v = buf[b, off >> 11, pl.ds(off & (W - 1), LANES)]
            plsc.addupdate_scatter(hist_ln, [lane_base | v], ones16)

        cp = nxt

    pltpu.sync_copy(hist_ln, out_hbm.at[wid])


@functools.cache
def _hist_sc():
    return pl.kernel(
        _hist_body,
        out_type=jax.ShapeDtypeStruct((NW, LANES * BINS), jnp.int32),
        mesh=plsc.VectorSubcoreMesh(core_axis_name="c", subcore_axis_name="s"),
        compiler_params=pltpu.CompilerParams(needs_layout_passes=False),
        scratch_types=[
            pltpu.VMEM((2, CROWS, W), jnp.int32),
            pltpu.VMEM((LANES * BINS,), jnp.int32),
            pltpu.SemaphoreType.DMA,
            pltpu.SemaphoreType.DMA,
        ],
    )


TILE = 1024


def _finish_body(part_ref, img_ref, thresh_ref, out_ref):
    step = pl.program_id(0)

    @pl.when(step == 0)
    def _():
        acc = part_ref[:, :BINS]
        for l in range(1, LANES):
            acc = acc + part_ref[:, l * BINS:(l + 1) * BINS]
        hist = jnp.sum(acc, axis=0, keepdims=True)
        vals = lax.broadcasted_iota(jnp.int32, (1, BINS), 1)
        fc = hist * vals
        rows = lax.broadcasted_iota(jnp.int32, (BINS, BINS), 0)
        cols = lax.broadcasted_iota(jnp.int32, (BINS, BINS), 1)
        mask = cols <= rows
        zero = jnp.zeros((BINS, BINS), jnp.int32)
        num_bk = jnp.sum(jnp.where(mask, hist, zero), axis=1,
                         keepdims=True)
        fc_bk = jnp.sum(jnp.where(mask, fc, zero), axis=1, keepdims=True)
        num_wh = jnp.sum(hist) - num_bk
        fc_wh = jnp.sum(fc) - fc_bk
        mean_bk = fc_bk / num_bk
        mean_wh = fc_wh / num_wh
        var = (num_bk.astype(jnp.float32) * num_wh.astype(jnp.float32)
               * (mean_bk - mean_wh) ** 2)
        var = jnp.where(jnp.isnan(var), jnp.float32(0), var)
        vmax = jnp.max(var)
        tidx = lax.broadcasted_iota(jnp.int32, (BINS, 1), 0)
        thresh_ref[0, 0] = jnp.min(
            jnp.where(var == vmax, tidx, jnp.int32(BINS)))

    t = thresh_ref[0, 0]
    out_ref[...] = jnp.where(img_ref[...] <= t, jnp.int32(0), jnp.int32(255))


_finish = pl.pallas_call(
    _finish_body,
    grid=(H // TILE,),
    in_specs=[
        pl.BlockSpec((NW, LANES * BINS), lambda i: (0, 0)),
        pl.BlockSpec((TILE, W), lambda i: (i, 0)),
    ],
    out_specs=[
        pl.BlockSpec(memory_space=pltpu.SMEM),
        pl.BlockSpec((TILE, W), lambda i: (i, 0)),
    ],
    out_shape=[
        jax.ShapeDtypeStruct((1, 1), jnp.int32),
        jax.ShapeDtypeStruct((H, W), jnp.int32),
    ],
    compiler_params=pltpu.CompilerParams(
        dimension_semantics=("arbitrary",)),
)


def kernel(img_HxW):
    partials = _hist_sc()(img_HxW)
    thresh2d, bin_img = _finish(partials, img_HxW)
    return thresh2d[0, 0], bin_img
